# baseline (speedup 1.0000x reference)
import functools

import jax
import jax.numpy as jnp
from jax import lax
from jax.experimental import pallas as pl
from jax.experimental.pallas import tpu as pltpu
from jax.experimental.pallas import tpu_sc as plsc

N = 10000
E = 320000
NP = 10240
NC = 2
NS = 16
NW = NC * NS
B = 128
GRP = 2
NSUP = 40
NB = NSUP * GRP
NB2 = NB + 2 * GRP
EPW = NB * B
RPT = NP // NS

_mesh = functools.partial(
    plsc.VectorSubcoreMesh, core_axis_name="c", subcore_axis_name="s")


def _deg_body(dst_hbm, ones_hbm, z_hbm, out_hbm, dst_v, ones_v, sem, acc):
    c = lax.axis_index("c")
    s = lax.axis_index("s")
    w = c * NS + s
    pltpu.sync_copy(dst_hbm.at[w], dst_v)
    pltpu.sync_copy(ones_hbm, ones_v)
    pltpu.sync_copy(z_hbm, acc.at[pl.ds(s * RPT, RPT)])
    plsc.subcore_barrier()
    def fire(b, carry):
        pltpu.async_copy(ones_v, acc.at[dst_v.at[b]], sem, add=True)
        return carry
    lax.fori_loop(0, NB2, fire, 0)
    def drain(b, carry):
        pltpu.make_async_copy(ones_v, acc.at[dst_v.at[0]], sem).wait()
        return carry
    lax.fori_loop(0, NB2, drain, 0)
    plsc.subcore_barrier()
    pltpu.sync_copy(acc.at[pl.ds(s * RPT, RPT)],
                    out_hbm.at[c, pl.ds(s * RPT, RPT)])


def _sc_degree(dst3, ones128, z128):
    return pl.kernel(
        _deg_body,
        out_type=jax.ShapeDtypeStruct((NC, NP, 128), jnp.float32),
        mesh=_mesh(),
        scratch_types=[
            pltpu.VMEM((NB2, B), jnp.int32),
            pltpu.VMEM((B, 128), jnp.float32),
            pltpu.SemaphoreType.DMA,
            pltpu.VMEM_SHARED((NP, 128), jnp.float32),
        ],
    )(dst3, ones128, z128)


def _agg_body(nslabs, *refs):
    xs = refs[:nslabs]
    src_hbm, dst_hbm, z_hbm = refs[nslabs:nslabs + 3]
    outs = refs[nslabs + 3:2 * nslabs + 3]
    rest = refs[2 * nslabs + 3:]
    si = rest[0:2]
    di = rest[2:4]
    rb = rest[4:4 + GRP]
    gsem = rest[4 + GRP:4 + 2 * GRP]
    ssem = rest[4 + 2 * GRP:4 + 3 * GRP]
    ism = rest[4 + 3 * GRP:6 + 3 * GRP]
    idm = rest[6 + 3 * GRP:8 + 3 * GRP]
    acc = rest[8 + 3 * GRP]
    c = lax.axis_index("c")
    s = lax.axis_index("s")
    w = c * NS + s

    def pref_s(sup, p):
        pltpu.async_copy(src_hbm.at[w, pl.ds(sup * GRP, GRP)], si[p], ism[p])

    def pref_d(sup, p):
        pltpu.async_copy(dst_hbm.at[w, pl.ds(sup * GRP, GRP)], di[p], idm[p])

    def wait_is(p):
        pltpu.make_async_copy(src_hbm.at[w, pl.ds(0, GRP)], si[p],
                              ism[p]).wait()

    def wait_id(p):
        pltpu.make_async_copy(dst_hbm.at[w, pl.ds(0, GRP)], di[p],
                              idm[p]).wait()

    for j in range(nslabs):
        pltpu.sync_copy(z_hbm, acc.at[pl.ds(s * RPT, RPT)])
        plsc.subcore_barrier()

        def gwait(u, j=j):
            pltpu.make_async_copy(xs[j].at[si[0].at[0]], rb[u],
                                  gsem[u]).wait()

        def swait(u):
            pltpu.make_async_copy(rb[u], acc.at[di[0].at[0]],
                                  ssem[u]).wait()

        pref_s(0, 0)
        pref_d(0, 0)
        pref_s(1, 1)
        pref_d(1, 1)
        wait_is(0)
        wait_id(0)
        for u in range(GRP):
            pltpu.async_copy(xs[j].at[si[0].at[u]], rb[u], gsem[u])

        def one_super(sup, p, j=j):
            p1 = 1 - p
            for u in range(GRP):
                gwait(u, j)
                pltpu.async_copy(rb[u], acc.at[di[p].at[u]], ssem[u],
                                 add=True)
            pref_s(sup + 2, p)
            for u in range(GRP):
                swait(u)
            pref_d(sup + 2, p)
            wait_is(p1)
            for u in range(GRP):
                pltpu.async_copy(xs[j].at[si[p1].at[u]], rb[u], gsem[u])
            wait_id(p1)

        def pair_step(q, carry, j=j):
            one_super(2 * q, 0, j)
            one_super(2 * q + 1, 1, j)
            return carry

        lax.fori_loop(0, NSUP // 2, pair_step, 0)
        for u in range(GRP):
            gwait(u, j)
        wait_is(1)
        wait_id(1)
        plsc.subcore_barrier()
        pltpu.sync_copy(acc.at[pl.ds(s * RPT, RPT)],
                        outs[j].at[c, pl.ds(s * RPT, RPT)])
        plsc.subcore_barrier()


def _sc_aggregate(slabs, src3, dst3, z128):
    k = len(slabs)
    return pl.kernel(
        functools.partial(_agg_body, k),
        out_type=[jax.ShapeDtypeStruct((NC, NP, 128), jnp.float32)] * k,
        mesh=_mesh(),
        scratch_types=(
            [pltpu.VMEM((GRP, B), jnp.int32)] * 4
            + [pltpu.VMEM((B, 128), jnp.float32)] * GRP
            + [pltpu.SemaphoreType.DMA] * (2 * GRP + 4)
            + [pltpu.VMEM_SHARED((NP, 128), jnp.float32)]
        ),
    )(*slabs, src3, dst3, z128)


def _prep_body(degp, x, xs_o, dinv_o):
    deg = degp[0] + degp[1] + 1.0
    dinv = lax.rsqrt(deg)[:, 0:1]
    xs_o[...] = x[...] * dinv
    dinv_o[...] = jnp.broadcast_to(dinv, dinv_o.shape)


def _tc_prep(degp, xp, rb=512):
    grid = (NP // rb,)
    return pl.pallas_call(
        _prep_body,
        grid=grid,
        in_specs=[
            pl.BlockSpec((NC, rb, 128), lambda i: (0, i, 0)),
            pl.BlockSpec((rb, 128), lambda i: (i, 0)),
        ],
        out_specs=[
            pl.BlockSpec((rb, 128), lambda i: (i, 0)),
            pl.BlockSpec((rb, 128), lambda i: (i, 0)),
        ],
        out_shape=[
            jax.ShapeDtypeStruct((NP, 128), jnp.float32),
            jax.ShapeDtypeStruct((NP, 128), jnp.float32),
        ],
    )(degp, xp)


def _l12_body(s1p, xs, dinvb, w1, b1, w2, o0, o1, o2, o3):
    dinv = dinvb[...][:, 0:1]
    agg0 = dinvb[...] * (s1p[0] + s1p[1] + xs[...])
    h1 = jnp.maximum(jnp.dot(agg0, w1[...],
                             preferred_element_type=jnp.float32) + b1[...], 0.0)
    t2 = jnp.dot(h1, w2[...], preferred_element_type=jnp.float32) * dinv
    o0[...] = t2[:, 0:128]
    o1[...] = t2[:, 128:256]
    o2[...] = t2[:, 256:384]
    o3[...] = t2[:, 384:512]


def _tc_layers12(s1p, xs, dinvb, w1, b1r, w2, rb=256):
    grid = (NP // rb,)
    return pl.pallas_call(
        _l12_body,
        grid=grid,
        in_specs=[
            pl.BlockSpec((NC, rb, 128), lambda i: (0, i, 0)),
            pl.BlockSpec((rb, 128), lambda i: (i, 0)),
            pl.BlockSpec((rb, 128), lambda i: (i, 0)),
            pl.BlockSpec((128, 1024), lambda i: (0, 0)),
            pl.BlockSpec((1, 1024), lambda i: (0, 0)),
            pl.BlockSpec((1024, 512), lambda i: (0, 0)),
        ],
        out_specs=[pl.BlockSpec((rb, 128), lambda i: (i, 0))] * 4,
        out_shape=[jax.ShapeDtypeStruct((NP, 128), jnp.float32)] * 4,
    )(s1p, xs, dinvb, w1, b1r, w2)


def _l3_body(p0, p1, p2, p3, t0, t1, t2, t3, dinvb, b2, w3, o):
    dinv = dinvb[...][:, 0:1]
    cols = [p0[0] + p0[1] + t0[...], p1[0] + p1[1] + t1[...],
            p2[0] + p2[1] + t2[...], p3[0] + p3[1] + t3[...]]
    s2 = jnp.concatenate(cols, axis=1)
    h2 = jnp.maximum(dinv * s2 + b2[...], 0.0)
    o[...] = jnp.dot(h2, w3[...], preferred_element_type=jnp.float32) * dinv


def _tc_layer3(s2ps, t2s, dinvb, b2r, w3, rb=256):
    grid = (NP // rb,)
    return pl.pallas_call(
        _l3_body,
        grid=grid,
        in_specs=(
            [pl.BlockSpec((NC, rb, 128), lambda i: (0, i, 0))] * 4
            + [pl.BlockSpec((rb, 128), lambda i: (i, 0))] * 4
            + [
                pl.BlockSpec((rb, 128), lambda i: (i, 0)),
                pl.BlockSpec((1, 512), lambda i: (0, 0)),
                pl.BlockSpec((512, 128), lambda i: (0, 0)),
            ]
        ),
        out_specs=pl.BlockSpec((rb, 128), lambda i: (i, 0)),
        out_shape=jax.ShapeDtypeStruct((NP, 128), jnp.float32),
    )(*s2ps, *t2s, dinvb, b2r, w3)


def _l4_body(s3p, t3s, dinvb, b3, wfc, bfc, o):
    dinv = dinvb[...][:, 0:1]
    h3 = jnp.maximum(dinv * (s3p[0] + s3p[1] + t3s[...]) + b3[...], 0.0)
    z = jnp.dot(h3, wfc[...], preferred_element_type=jnp.float32) + bfc[...]
    m = jnp.max(z, axis=1, keepdims=True)
    lse = m + jnp.log(jnp.sum(jnp.exp(z - m), axis=1, keepdims=True))
    o[...] = z - lse


def _tc_layer4(s3p, t3s, dinvb, b3r, wfc, bfcr, rb=256):
    grid = (NP // rb,)
    return pl.pallas_call(
        _l4_body,
        grid=grid,
        in_specs=[
            pl.BlockSpec((NC, rb, 128), lambda i: (0, i, 0)),
            pl.BlockSpec((rb, 128), lambda i: (i, 0)),
            pl.BlockSpec((rb, 128), lambda i: (i, 0)),
            pl.BlockSpec((1, 128), lambda i: (0, 0)),
            pl.BlockSpec((128, 64), lambda i: (0, 0)),
            pl.BlockSpec((1, 64), lambda i: (0, 0)),
        ],
        out_specs=pl.BlockSpec((rb, 64), lambda i: (i, 0)),
        out_shape=jax.ShapeDtypeStruct((NP, 64), jnp.float32),
    )(s3p, t3s, dinvb, b3r, wfc, bfcr)


def kernel(x, edge_index, W1, b1, W2, b2, W3, b3, Wfc, bfc):
    xp = jnp.pad(x, ((0, NP - N), (0, 0)))

    def pack(e):
        flat = jnp.concatenate(
            [e, jnp.full((NW * EPW - E,), N, dtype=jnp.int32)])
        main = flat.reshape(NW, NB, B)
        tail = jnp.full((NW, NB2 - NB, B), N, dtype=jnp.int32)
        return jnp.concatenate([main, tail], axis=1)

    src3 = pack(edge_index[0])
    dst3 = pack(edge_index[1])
    z128 = jnp.zeros((RPT, 128), jnp.float32)
    ones128 = jnp.ones((B, 128), jnp.float32)
    b1r = b1.reshape(1, 1024)
    b2r = b2.reshape(1, 512)
    b3r = b3.reshape(1, 128)
    bfcr = bfc.reshape(1, 64)

    degp = _sc_degree(dst3, ones128, z128)
    xs, dinvb = _tc_prep(degp, xp)
    (s1p,) = _sc_aggregate([xs], src3, dst3, z128)
    t2s = _tc_layers12(s1p, xs, dinvb, W1, b1r, W2)
    s2ps = _sc_aggregate(list(t2s), src3, dst3, z128)
    t3s = _tc_layer3(s2ps, t2s, dinvb, b2r, W3)
    (s3p,) = _sc_aggregate([t3s], src3, dst3, z128)
    out = _tc_layer4(s3p, t3s, dinvb, b3r, Wfc, bfcr)
    return out[:N]


# speedup vs baseline: 1.3746x; 1.3746x over previous
import functools

import jax
import jax.numpy as jnp
from jax import lax
from jax.experimental import pallas as pl
from jax.experimental.pallas import tpu as pltpu
from jax.experimental.pallas import tpu_sc as plsc

N = 10000
E = 320000
NP = 10240
NC = 2
NS = 16
NW = NC * NS
B = 128
NB = 80
NB2 = NB + 4
EPW = NB * B
RPT = NP // NS

_mesh = functools.partial(
    plsc.VectorSubcoreMesh, core_axis_name="c", subcore_axis_name="s")


def _unpack_batch(pk, b, srcb, dstb):
    for t in range(8):
        v = pk[b, pl.ds(16 * t, 16)]
        if srcb is not None:
            srcb[pl.ds(16 * t, 16)] = v & 0xFFFF
        dstb[pl.ds(16 * t, 16)] = lax.shift_right_logical(v, 16)


def _deg_body(pk_hbm, ones_hbm, z_hbm, out_hbm, pk, dstb, ones_v, acc):
    c = lax.axis_index("c")
    s = lax.axis_index("s")
    w = c * NS + s
    pltpu.sync_copy(pk_hbm.at[w], pk)
    pltpu.sync_copy(ones_hbm, ones_v)
    pltpu.sync_copy(z_hbm, acc.at[pl.ds(s * RPT, RPT)])
    plsc.subcore_barrier()
    def step(b, carry):
        _unpack_batch(pk, b, None, dstb)
        pltpu.sync_copy(ones_v, acc.at[dstb], add=True)
        return carry
    lax.fori_loop(0, NB, step, 0)
    plsc.subcore_barrier()
    pltpu.sync_copy(acc.at[pl.ds(s * RPT, RPT)],
                    out_hbm.at[c, pl.ds(s * RPT, RPT)])


def _sc_degree(pk3, ones128, z128):
    return pl.kernel(
        _deg_body,
        out_type=jax.ShapeDtypeStruct((NC, NP, 128), jnp.float32),
        mesh=_mesh(),
        scratch_types=[
            pltpu.VMEM((NB2, B), jnp.int32),
            pltpu.VMEM((B,), jnp.int32),
            pltpu.VMEM((B, 128), jnp.float32),
            pltpu.VMEM_SHARED((NP, 128), jnp.float32),
        ],
    )(pk3, ones128, z128)


def _agg_body(nslabs, *refs):
    xs = refs[:nslabs]
    pk_hbm, z_hbm = refs[nslabs:nslabs + 2]
    outs = refs[nslabs + 2:2 * nslabs + 2]
    rest = refs[2 * nslabs + 2:]
    pk = rest[0]
    srcb = rest[1:3]
    dstb = rest[3:5]
    rb = rest[5:7]
    gsem = rest[7:9]
    acc = rest[9]
    c = lax.axis_index("c")
    s = lax.axis_index("s")
    w = c * NS + s
    pltpu.sync_copy(pk_hbm.at[w], pk)
    for j in range(nslabs):
        pltpu.sync_copy(z_hbm, acc.at[pl.ds(s * RPT, RPT)])
        plsc.subcore_barrier()
        _unpack_batch(pk, 0, srcb[0], dstb[0])
        pltpu.async_copy(xs[j].at[srcb[0]], rb[0], gsem[0])

        def pair(q, carry, j=j):
            for k in (0, 1):
                b = 2 * q + k
                _unpack_batch(pk, b + 1, srcb[1 - k], dstb[1 - k])
                pltpu.async_copy(xs[j].at[srcb[1 - k]], rb[1 - k],
                                 gsem[1 - k])
                pltpu.make_async_copy(xs[j].at[srcb[k]], rb[k],
                                      gsem[k]).wait()
                pltpu.sync_copy(rb[k], acc.at[dstb[k]], add=True)
            return carry

        lax.fori_loop(0, NB // 2, pair, 0)
        pltpu.make_async_copy(xs[j].at[srcb[0]], rb[0], gsem[0]).wait()
        plsc.subcore_barrier()
        pltpu.sync_copy(acc.at[pl.ds(s * RPT, RPT)],
                        outs[j].at[c, pl.ds(s * RPT, RPT)])
        plsc.subcore_barrier()


def _sc_aggregate(slabs, pk3, z128):
    k = len(slabs)
    return pl.kernel(
        functools.partial(_agg_body, k),
        out_type=[jax.ShapeDtypeStruct((NC, NP, 128), jnp.float32)] * k,
        mesh=_mesh(),
        scratch_types=(
            [pltpu.VMEM((NB2, B), jnp.int32)]
            + [pltpu.VMEM((B,), jnp.int32)] * 4
            + [pltpu.VMEM((B, 128), jnp.float32)] * 2
            + [pltpu.SemaphoreType.DMA] * 2
            + [pltpu.VMEM_SHARED((NP, 128), jnp.float32)]
        ),
    )(*slabs, pk3, z128)


def _prep_body(degp, x, xs_o, dinv_o):
    deg = degp[0] + degp[1] + 1.0
    dinv = lax.rsqrt(deg)[:, 0:1]
    xs_o[...] = x[...] * dinv
    dinv_o[...] = jnp.broadcast_to(dinv, dinv_o.shape)


def _tc_prep(degp, xp, rb=512):
    grid = (NP // rb,)
    return pl.pallas_call(
        _prep_body,
        grid=grid,
        in_specs=[
            pl.BlockSpec((NC, rb, 128), lambda i: (0, i, 0)),
            pl.BlockSpec((rb, 128), lambda i: (i, 0)),
        ],
        out_specs=[
            pl.BlockSpec((rb, 128), lambda i: (i, 0)),
            pl.BlockSpec((rb, 128), lambda i: (i, 0)),
        ],
        out_shape=[
            jax.ShapeDtypeStruct((NP, 128), jnp.float32),
            jax.ShapeDtypeStruct((NP, 128), jnp.float32),
        ],
    )(degp, xp)


def _l12_body(s1p, xs, dinvb, w1, b1, w2, o0, o1, o2, o3):
    dinv = dinvb[...][:, 0:1]
    agg0 = dinvb[...] * (s1p[0] + s1p[1] + xs[...])
    h1 = jnp.maximum(jnp.dot(agg0, w1[...],
                             preferred_element_type=jnp.float32) + b1[...], 0.0)
    t2 = jnp.dot(h1, w2[...], preferred_element_type=jnp.float32) * dinv
    o0[...] = t2[:, 0:128]
    o1[...] = t2[:, 128:256]
    o2[...] = t2[:, 256:384]
    o3[...] = t2[:, 384:512]


def _tc_layers12(s1p, xs, dinvb, w1, b1r, w2, rb=256):
    grid = (NP // rb,)
    return pl.pallas_call(
        _l12_body,
        grid=grid,
        in_specs=[
            pl.BlockSpec((NC, rb, 128), lambda i: (0, i, 0)),
            pl.BlockSpec((rb, 128), lambda i: (i, 0)),
            pl.BlockSpec((rb, 128), lambda i: (i, 0)),
            pl.BlockSpec((128, 1024), lambda i: (0, 0)),
            pl.BlockSpec((1, 1024), lambda i: (0, 0)),
            pl.BlockSpec((1024, 512), lambda i: (0, 0)),
        ],
        out_specs=[pl.BlockSpec((rb, 128), lambda i: (i, 0))] * 4,
        out_shape=[jax.ShapeDtypeStruct((NP, 128), jnp.float32)] * 4,
    )(s1p, xs, dinvb, w1, b1r, w2)


def _l3_body(p0, p1, p2, p3, t0, t1, t2, t3, dinvb, b2, w3, o):
    dinv = dinvb[...][:, 0:1]
    cols = [p0[0] + p0[1] + t0[...], p1[0] + p1[1] + t1[...],
            p2[0] + p2[1] + t2[...], p3[0] + p3[1] + t3[...]]
    s2 = jnp.concatenate(cols, axis=1)
    h2 = jnp.maximum(dinv * s2 + b2[...], 0.0)
    o[...] = jnp.dot(h2, w3[...], preferred_element_type=jnp.float32) * dinv


def _tc_layer3(s2ps, t2s, dinvb, b2r, w3, rb=256):
    grid = (NP // rb,)
    return pl.pallas_call(
        _l3_body,
        grid=grid,
        in_specs=(
            [pl.BlockSpec((NC, rb, 128), lambda i: (0, i, 0))] * 4
            + [pl.BlockSpec((rb, 128), lambda i: (i, 0))] * 4
            + [
                pl.BlockSpec((rb, 128), lambda i: (i, 0)),
                pl.BlockSpec((1, 512), lambda i: (0, 0)),
                pl.BlockSpec((512, 128), lambda i: (0, 0)),
            ]
        ),
        out_specs=pl.BlockSpec((rb, 128), lambda i: (i, 0)),
        out_shape=jax.ShapeDtypeStruct((NP, 128), jnp.float32),
    )(*s2ps, *t2s, dinvb, b2r, w3)


def _l4_body(s3p, t3s, dinvb, b3, wfc, bfc, o):
    dinv = dinvb[...][:, 0:1]
    h3 = jnp.maximum(dinv * (s3p[0] + s3p[1] + t3s[...]) + b3[...], 0.0)
    z = jnp.dot(h3, wfc[...], preferred_element_type=jnp.float32) + bfc[...]
    m = jnp.max(z, axis=1, keepdims=True)
    lse = m + jnp.log(jnp.sum(jnp.exp(z - m), axis=1, keepdims=True))
    o[...] = z - lse


def _tc_layer4(s3p, t3s, dinvb, b3r, wfc, bfcr, rb=256):
    grid = (NP // rb,)
    return pl.pallas_call(
        _l4_body,
        grid=grid,
        in_specs=[
            pl.BlockSpec((NC, rb, 128), lambda i: (0, i, 0)),
            pl.BlockSpec((rb, 128), lambda i: (i, 0)),
            pl.BlockSpec((rb, 128), lambda i: (i, 0)),
            pl.BlockSpec((1, 128), lambda i: (0, 0)),
            pl.BlockSpec((128, 64), lambda i: (0, 0)),
            pl.BlockSpec((1, 64), lambda i: (0, 0)),
        ],
        out_specs=pl.BlockSpec((rb, 64), lambda i: (i, 0)),
        out_shape=jax.ShapeDtypeStruct((NP, 64), jnp.float32),
    )(s3p, t3s, dinvb, b3r, wfc, bfcr)


def kernel(x, edge_index, W1, b1, W2, b2, W3, b3, Wfc, bfc):
    xp = jnp.pad(x, ((0, NP - N), (0, 0)))

    def pack(e):
        flat = jnp.concatenate(
            [e, jnp.full((NW * EPW - E,), N, dtype=jnp.int32)])
        main = flat.reshape(NW, NB, B)
        tail = jnp.full((NW, NB2 - NB, B), N, dtype=jnp.int32)
        return jnp.concatenate([main, tail], axis=1)

    pk3 = pack(edge_index[0]) | (pack(edge_index[1]) << 16)
    z128 = jnp.zeros((RPT, 128), jnp.float32)
    ones128 = jnp.ones((B, 128), jnp.float32)
    b1r = b1.reshape(1, 1024)
    b2r = b2.reshape(1, 512)
    b3r = b3.reshape(1, 128)
    bfcr = bfc.reshape(1, 64)

    degp = _sc_degree(pk3, ones128, z128)
    xs, dinvb = _tc_prep(degp, xp)
    (s1p,) = _sc_aggregate([xs], pk3, z128)
    t2s = _tc_layers12(s1p, xs, dinvb, W1, b1r, W2)
    s2ps = _sc_aggregate(list(t2s), pk3, z128)
    t3s = _tc_layer3(s2ps, t2s, dinvb, b2r, W3)
    (s3p,) = _sc_aggregate([t3s], pk3, z128)
    out = _tc_layer4(s3p, t3s, dinvb, b3r, Wfc, bfcr)
    return out[:N]
